# baseline (speedup 1.0000x reference)
import functools

import jax
import jax.numpy as jnp
from jax import lax
from jax.experimental import pallas as pl
from jax.experimental.pallas import tpu as pltpu
from jax.experimental.pallas import tpu_sc as plsc

B = 16384
L = 20
D = 32
V = 1000000
T8 = V // 8
NW = 32
BPW = B // NW
CHUNK = 128
NCHUNK = BPW // CHUNK
CV = CHUNK * L


def _make_sc_kernel():
    mesh = plsc.VectorSubcoreMesh(core_axis_name="c", subcore_axis_name="s")

    @functools.partial(
        pl.kernel,
        mesh=mesh,
        out_type=jax.ShapeDtypeStruct((B * L,), jnp.float32),
        compiler_params=pltpu.CompilerParams(
            use_tc_tiling_on_sc=False, needs_layout_passes=False),
        scratch_types=[
            pltpu.VMEM((CHUNK,), jnp.int32),
            pltpu.VMEM((CV,), jnp.int32),
            pltpu.VMEM((8 * CHUNK,), jnp.int32),
            pltpu.VMEM((8 * CV,), jnp.int32),
            pltpu.VMEM((CHUNK, D), jnp.float32),
            pltpu.VMEM((CV, D), jnp.float32),
            pltpu.VMEM((CV,), jnp.float32),
            pltpu.SemaphoreType.DMA,
        ],
    )
    def k(center_hbm, ctx_hbm, u_hbm, v_hbm, out_hbm,
          cidx_v, ctxidx_v, uidx_v, vidx_v, u_rows, v_rows, out_v, sem):
        wid = lax.axis_index("s") * 2 + lax.axis_index("c")
        lanes = lax.iota(jnp.int32, 16)
        zero = jnp.zeros((16,), jnp.float32)

        for c in range(NCHUNK):
            base = wid * BPW + c * CHUNK
            pltpu.sync_copy(center_hbm.at[pl.ds(base, CHUNK)], cidx_v)
            pltpu.sync_copy(ctx_hbm.at[pl.ds(base * L, CV)], ctxidx_v)

            def build(src, dst, n):
                def body(i, carry):
                    pos = i * 16 + lanes
                    r = plsc.load_gather(src, [pos])
                    t = jnp.right_shift(r, 3)
                    res = jnp.bitwise_and(r, 7)
                    for kk in range(8):
                        plsc.store_scatter(dst, [kk * n + pos],
                                           jnp.where(res == kk, t, -1))
                    return carry
                lax.fori_loop(0, n // 16, body, 0)

            build(cidx_v, uidx_v, CHUNK)
            build(ctxidx_v, vidx_v, CV)

            copies = []
            for kk in range(8):
                copies.append(pltpu.async_copy(
                    u_hbm.at[kk].at[
                        plsc.Indices(uidx_v.at[pl.ds(kk * CHUNK, CHUNK)],
                                     ignored_value=-1)],
                    u_rows, sem))
                for jj in range(CV // 128):
                    copies.append(pltpu.async_copy(
                        v_hbm.at[kk].at[
                            plsc.Indices(
                                vidx_v.at[pl.ds(kk * CV + jj * 128, 128)],
                                ignored_value=-1)],
                        v_rows.at[pl.ds(jj * 128, 128)], sem))
            for cp in copies:
                cp.wait()

            def g_body(g, carry):
                row_u = g * 16 + lanes
                for lp in range(L // 4):
                    rows_v = [row_u * L + (lp * 4 + q) for q in range(4)]

                    def d_body(dd, accs):
                        dcol = jnp.full((16,), dd, jnp.int32)
                        uu = plsc.load_gather(u_rows, [row_u, dcol])
                        return tuple(
                            acc + plsc.load_gather(v_rows, [rows_v[q], dcol]) * uu
                            for q, acc in enumerate(accs))

                    accs = lax.fori_loop(0, D, d_body, (zero, zero, zero, zero))
                    for q in range(4):
                        plsc.store_scatter(out_v, [rows_v[q]], accs[q])
                return carry

            lax.fori_loop(0, CHUNK // 16, g_body, 0)
            pltpu.sync_copy(out_v, out_hbm.at[pl.ds(base * L, CV)])

    return k


_sc_kernel = _make_sc_kernel()


def kernel(center_words, context_words, u_table, v_table):
    center = center_words.astype(jnp.int32)
    ctx_flat = context_words.astype(jnp.int32).reshape(-1)
    u3 = u_table.reshape(T8, 8, D).transpose(1, 0, 2)
    v3 = v_table.reshape(T8, 8, D).transpose(1, 0, 2)
    out_flat = _sc_kernel(center, ctx_flat, u3, v3)
    return out_flat.reshape(B, L)


# speedup vs baseline: 1.5315x; 1.5315x over previous
import functools

import jax
import jax.numpy as jnp
from jax import lax
from jax.experimental import pallas as pl
from jax.experimental.pallas import tpu as pltpu
from jax.experimental.pallas import tpu_sc as plsc

B = 16384
L = 20
D = 32
V = 1000000
NW = 32
BPW = B // NW
CHUNK = 64
NCHUNK = BPW // CHUNK
CV = CHUNK * L

DEPAD_BLK = 5000


def _depad_body(i_ref, o_ref):
    x = i_ref[...]
    merged = jnp.concatenate([x[:, s, :] for s in range(8)], axis=-1)
    o_ref[...] = merged.reshape(DEPAD_BLK * 8 * D)


_depad = pl.pallas_call(
    _depad_body,
    grid=(V // 8 // DEPAD_BLK,),
    in_specs=[pl.BlockSpec((DEPAD_BLK, 8, D), lambda i: (i, 0, 0))],
    out_specs=pl.BlockSpec((DEPAD_BLK * 8 * D,), lambda i: (i,)),
    out_shape=jax.ShapeDtypeStruct((V * D,), jnp.float32),
)


def _make_sc_kernel():
    mesh = plsc.VectorSubcoreMesh(core_axis_name="c", subcore_axis_name="s")

    @functools.partial(
        pl.kernel,
        mesh=mesh,
        out_type=jax.ShapeDtypeStruct((B * L,), jnp.float32),
        compiler_params=pltpu.CompilerParams(
            use_tc_tiling_on_sc=False, needs_layout_passes=False),
        scratch_types=[
            pltpu.VMEM((2, CHUNK), jnp.int32),
            pltpu.VMEM((2, CV), jnp.int32),
            pltpu.VMEM((2, CHUNK, D), jnp.float32),
            pltpu.VMEM((2, CV, D), jnp.float32),
            pltpu.VMEM((CV,), jnp.float32),
            pltpu.SemaphoreType.DMA,
            pltpu.SemaphoreType.DMA,
        ],
    )
    def k(center_hbm, ctx_hbm, u_hbm, v_hbm, out_hbm,
          cidx_v, ctxidx_v, u_rows, v_rows, out_v, sem0, sem1):
        wid = lax.axis_index("s") * 2 + lax.axis_index("c")
        lanes = lax.iota(jnp.int32, 16)
        zero = jnp.zeros((16,), jnp.float32)
        sems = (sem0, sem1)

        def fire(c, p):
            base = wid * BPW + c * CHUNK
            pltpu.sync_copy(center_hbm.at[pl.ds(base, CHUNK)], cidx_v.at[p])
            pltpu.sync_copy(ctx_hbm.at[pl.ds(base * L, CV)], ctxidx_v.at[p])
            copies = [pltpu.async_copy(
                u_hbm.at[cidx_v.at[p]], u_rows.at[p], sems[p])]
            for j in range(CV // 128):
                copies.append(pltpu.async_copy(
                    v_hbm.at[ctxidx_v.at[p, pl.ds(j * 128, 128)]],
                    v_rows.at[p, pl.ds(j * 128, 128)], sems[p]))
            return copies

        inflight = fire(0, 0)
        for c in range(NCHUNK):
            p = c % 2
            for cp in inflight:
                cp.wait()
            if c + 1 < NCHUNK:
                inflight = fire(c + 1, (c + 1) % 2)

            def g_body(g, carry):
                row_u = g * 16 + lanes
                for lp in range(L // 4):
                    rows_v = [row_u * L + (lp * 4 + q) for q in range(4)]

                    def d_body(dd, accs):
                        dcol = jnp.full((16,), dd, jnp.int32)
                        uu = plsc.load_gather(u_rows.at[p], [row_u, dcol])
                        return tuple(
                            acc + plsc.load_gather(
                                v_rows.at[p], [rows_v[q], dcol]) * uu
                            for q, acc in enumerate(accs))

                    accs = lax.fori_loop(0, D, d_body, (zero, zero, zero, zero))
                    for q in range(4):
                        plsc.store_scatter(out_v, [rows_v[q]], accs[q])
                return carry

            lax.fori_loop(0, CHUNK // 16, g_body, 0)
            base = wid * BPW + c * CHUNK
            pltpu.sync_copy(out_v, out_hbm.at[pl.ds(base * L, CV)])

    return k


_sc_kernel = _make_sc_kernel()


def kernel(center_words, context_words, u_table, v_table):
    center = center_words.astype(jnp.int32)
    ctx_flat = context_words.astype(jnp.int32).reshape(-1)
    u_lin = _depad(u_table.reshape(V // 8, 8, D)).reshape(V, D)
    v_lin = _depad(v_table.reshape(V // 8, 8, D)).reshape(V, D)
    out_flat = _sc_kernel(center, ctx_flat, u_lin, v_lin)
    return out_flat.reshape(B, L)
